# baseline (speedup 1.0000x reference)
import functools

import jax
import jax.numpy as jnp
from jax import lax
from jax.experimental import pallas as pl
from jax.experimental.pallas import tpu as pltpu
from jax.experimental.pallas import tpu_sc as plsc

_NALLELES = 10
_D = 32
_L = 16
_NC = 2
_NS = 16
_NW = _NC * _NS
_NB = 3


def _sc_embed(packed, table, n_rows, chunk):
  per_w = n_rows // _NW
  n_chunks = per_w // chunk
  assert per_w % chunk == 0
  n_triples = (n_chunks - 8) // _NB
  assert n_triples * _NB == n_chunks - 8 and n_triples >= 1

  mesh = plsc.VectorSubcoreMesh(core_axis_name="c", subcore_axis_name="s")

  @functools.partial(
      pl.kernel,
      mesh=mesh,
      out_type=jax.ShapeDtypeStruct((n_rows, _D), jnp.float32),
      compiler_params=pltpu.CompilerParams(use_tc_tiling_on_sc=False),
      scratch_types=(
          [pltpu.VMEM((chunk,), jnp.int32)] * _NB
          + [pltpu.VMEM((chunk,), jnp.int32)] * _NB
          + [pltpu.VMEM((chunk,), jnp.int32)] * _NB
          + [pltpu.VMEM((chunk, _D), jnp.float32)] * _NB
          + [pltpu.SemaphoreType.DMA] * (4 * _NB)),
  )
  def k(w_hbm, table_hbm, out_hbm,
        w0, w1, w2, x0, x1, x2, y0, y1, y2, o0, o1, o2,
        is0, is1, is2, g0s0, g0s1, g0s2, g1s0, g1s1, g1s2,
        ws0, ws1, ws2):
    w_v = (w0, w1, w2)
    ix0_v = (x0, x1, x2)
    ix1_v = (y0, y1, y2)
    o_v = (o0, o1, o2)
    isem = (is0, is1, is2)
    g0sem = (g0s0, g0s1, g0s2)
    g1sem = (g1s0, g1s1, g1s2)
    wsem = (ws0, ws1, ws2)

    wid = lax.axis_index("s") * _NC + lax.axis_index("c")
    w_base = wid * per_w

    def issue_in(t, r):
      base = w_base + t * chunk
      pltpu.async_copy(w_hbm.at[pl.ds(base, chunk)], w_v[r], isem[r])

    def wait_in(r):
      pltpu.make_async_copy(w_hbm.at[pl.ds(0, chunk)], w_v[r],
                            isem[r]).wait()

    def compute_idx(r):
      def body(j, _):
        s = j * _L
        w = w_v[r][pl.ds(s, _L)]
        p = lax.shift_right_logical(w, 8) * _NALLELES
        ix0_v[r][pl.ds(s, _L)] = (
            p + (lax.shift_right_logical(w, 4) & 15))
        ix1_v[r][pl.ds(s, _L)] = p + (w & 15)
        return 0

      lax.fori_loop(0, chunk // _L, body, 0, unroll=4)

    def issue_g0(r):
      pltpu.async_copy(table_hbm.at[ix0_v[r]], o_v[r], g0sem[r])

    def wait_g0(r):
      pltpu.make_async_copy(table_hbm.at[pl.ds(0, chunk)], o_v[r],
                            g0sem[r]).wait()

    def issue_g1(r):
      pltpu.async_copy(table_hbm.at[ix1_v[r]], o_v[r], g1sem[r], add=True)

    def wait_g1(r):
      pltpu.make_async_copy(table_hbm.at[pl.ds(0, chunk)], o_v[r],
                            g1sem[r]).wait()

    def issue_wb(t, r):
      base = w_base + t * chunk
      pltpu.async_copy(o_v[r], out_hbm.at[pl.ds(base, chunk)], wsem[r])

    def wait_wb(r):
      pltpu.make_async_copy(o_v[r], out_hbm.at[pl.ds(0, chunk)],
                            wsem[r]).wait()

    def step(t, r, do_in=True, do_g1=True, do_wb=True, do_wbwait=True):
      r1 = (r + 2) % _NB
      r2 = (r + 1) % _NB
      wait_in(r)
      compute_idx(r)
      if do_in:
        issue_in(t + _NB, r)
      if do_g1:
        wait_g0(r1)
        issue_g1(r1)
      if do_wb:
        wait_g1(r2)
        issue_wb(t - 2, r2)
      if do_wbwait:
        wait_wb(r)
      issue_g0(r)

    issue_in(0, 0)
    issue_in(1, 1)
    issue_in(2, 2)
    step(0, 0, do_g1=False, do_wb=False, do_wbwait=False)
    step(1, 1, do_wb=False, do_wbwait=False)
    step(2, 2, do_wbwait=False)

    def triple(g, _):
      for r in range(_NB):
        step(_NB * g + r, r)
      return 0

    lax.fori_loop(1, n_triples + 1, triple, 0)

    tt = n_chunks - 5
    step(tt, tt % _NB)
    step(tt + 1, (tt + 1) % _NB)
    step(tt + 2, (tt + 2) % _NB, do_in=False)
    step(tt + 3, (tt + 3) % _NB, do_in=False)
    step(tt + 4, (tt + 4) % _NB, do_in=False)

    tl = n_chunks - 1
    rl = tl % _NB
    rp = (tl - 1) % _NB
    wait_g0(rl)
    issue_g1(rl)
    wait_g1(rp)
    issue_wb(tl - 1, rp)
    wait_g1(rl)
    issue_wb(tl, rl)
    wait_wb((tl + 1) % _NB)
    wait_wb(rp)
    wait_wb(rl)

  return k(packed, table)


def kernel(alleles, positions, table):
  b, s, _ = alleles.shape
  n = b * s
  packed = (
      lax.shift_left(positions.astype(jnp.int32), 8)
      | lax.shift_left(alleles[:, :, 0].astype(jnp.int32), 4)
      | alleles[:, :, 1].astype(jnp.int32)
  ).reshape(n)
  out = _sc_embed(packed, table, n, 512)
  return out.reshape(b, s, _D)


# speedup vs baseline: 1.0030x; 1.0030x over previous
import functools

import jax
import jax.numpy as jnp
from jax import lax
from jax.experimental import pallas as pl
from jax.experimental.pallas import tpu as pltpu
from jax.experimental.pallas import tpu_sc as plsc

_NALLELES = 10
_D = 32
_L = 16
_NC = 2
_NS = 16
_NW = _NC * _NS
_NB = 3


def _sc_embed(packed, table, n_rows, chunk):
  per_w = n_rows // _NW
  n_chunks = per_w // chunk
  assert per_w % chunk == 0
  n_triples = (n_chunks - 8) // _NB
  assert n_triples * _NB == n_chunks - 8 and n_triples >= 1

  mesh = plsc.VectorSubcoreMesh(core_axis_name="c", subcore_axis_name="s")

  @functools.partial(
      pl.kernel,
      mesh=mesh,
      out_type=jax.ShapeDtypeStruct((n_rows, _D), jnp.float32),
      compiler_params=pltpu.CompilerParams(use_tc_tiling_on_sc=False),
      scratch_types=(
          [pltpu.VMEM((chunk,), jnp.int32)] * _NB
          + [pltpu.VMEM((chunk,), jnp.int32)] * _NB
          + [pltpu.VMEM((chunk,), jnp.int32)] * _NB
          + [pltpu.VMEM((chunk, _D), jnp.float32)] * _NB
          + [pltpu.SemaphoreType.DMA] * (4 * _NB)),
  )
  def k(w_hbm, table_hbm, out_hbm,
        w0, w1, w2, x0, x1, x2, y0, y1, y2, o0, o1, o2,
        is0, is1, is2, g0s0, g0s1, g0s2, g1s0, g1s1, g1s2,
        ws0, ws1, ws2):
    w_v = (w0, w1, w2)
    ix0_v = (x0, x1, x2)
    ix1_v = (y0, y1, y2)
    o_v = (o0, o1, o2)
    isem = (is0, is1, is2)
    g0sem = (g0s0, g0s1, g0s2)
    g1sem = (g1s0, g1s1, g1s2)
    wsem = (ws0, ws1, ws2)

    wid = lax.axis_index("s") * _NC + lax.axis_index("c")
    w_base = wid * per_w

    def issue_in(t, r):
      base = w_base + t * chunk
      pltpu.async_copy(w_hbm.at[pl.ds(base, chunk)], w_v[r], isem[r])

    def wait_in(r):
      pltpu.make_async_copy(w_hbm.at[pl.ds(0, chunk)], w_v[r],
                            isem[r]).wait()

    def compute_idx(r):
      def body(j, _):
        s = j * _L
        w = w_v[r][pl.ds(s, _L)]
        p = lax.shift_right_logical(w, 8) * _NALLELES
        ix0_v[r][pl.ds(s, _L)] = (
            p + (lax.shift_right_logical(w, 4) & 15))
        ix1_v[r][pl.ds(s, _L)] = p + (w & 15)
        return 0

      lax.fori_loop(0, chunk // _L, body, 0, unroll=4)

    def issue_g0(r):
      pltpu.async_copy(table_hbm.at[ix0_v[r]], o_v[r], g0sem[r])

    def wait_g0(r):
      pltpu.make_async_copy(table_hbm.at[pl.ds(0, chunk)], o_v[r],
                            g0sem[r]).wait()

    def issue_g1(r):
      pltpu.async_copy(table_hbm.at[ix1_v[r]], o_v[r], g1sem[r], add=True)

    def wait_g1(r):
      pltpu.make_async_copy(table_hbm.at[pl.ds(0, chunk)], o_v[r],
                            g1sem[r]).wait()

    def issue_wb(t, r):
      base = w_base + t * chunk
      pltpu.async_copy(o_v[r], out_hbm.at[pl.ds(base, chunk)], wsem[r])

    def wait_wb(r):
      pltpu.make_async_copy(o_v[r], out_hbm.at[pl.ds(0, chunk)],
                            wsem[r]).wait()

    def step(t, r, do_in=True, do_g1=True, do_wb=True, do_wbwait=True):
      r1 = (r + 2) % _NB
      r2 = (r + 1) % _NB
      wait_in(r)
      compute_idx(r)
      if do_in:
        issue_in(t + _NB, r)
      if do_g1:
        wait_g0(r1)
        issue_g1(r1)
      if do_wb:
        wait_g1(r2)
        issue_wb(t - 2, r2)
      if do_wbwait:
        wait_wb(r)
      issue_g0(r)

    issue_in(0, 0)
    issue_in(1, 1)
    issue_in(2, 2)
    step(0, 0, do_g1=False, do_wb=False, do_wbwait=False)
    step(1, 1, do_wb=False, do_wbwait=False)
    step(2, 2, do_wbwait=False)

    def triple(g, _):
      for r in range(_NB):
        step(_NB * g + r, r)
      return 0

    lax.fori_loop(1, n_triples + 1, triple, 0)

    tt = n_chunks - 5
    step(tt, tt % _NB)
    step(tt + 1, (tt + 1) % _NB)
    step(tt + 2, (tt + 2) % _NB, do_in=False)
    step(tt + 3, (tt + 3) % _NB, do_in=False)
    step(tt + 4, (tt + 4) % _NB, do_in=False)

    tl = n_chunks - 1
    rl = tl % _NB
    rp = (tl - 1) % _NB
    wait_g0(rl)
    issue_g1(rl)
    wait_g1(rp)
    issue_wb(tl - 1, rp)
    wait_g1(rl)
    issue_wb(tl, rl)
    wait_wb((tl + 1) % _NB)
    wait_wb(rp)
    wait_wb(rl)

  return k(packed, table)


def kernel(alleles, positions, table):
  b, s, _ = alleles.shape
  n = b * s
  packed = (
      lax.shift_left(positions.astype(jnp.int32), 8)
      | lax.shift_left(alleles[:, :, 0].astype(jnp.int32), 4)
      | alleles[:, :, 1].astype(jnp.int32)
  ).reshape(n)
  out = _sc_embed(packed, table, n, 800)
  return out.reshape(b, s, _D)
